# baseline (speedup 1.0000x reference)
import functools

import jax
import jax.numpy as jnp
from jax import lax
from jax.experimental import pallas as pl
from jax.experimental.pallas import tpu as pltpu
from jax.experimental.pallas import tpu_sc as plsc

N = 10000
E = 320000
D = 128
EMB = 128
B = 8192

NC = 2
NS = 16
NW = NC * NS

EPW = 10240
EPAD = NW * EPW
Q = 40
SEG = 64
NSEG = EPW // (SEG * Q)
SINK = N
NPR = N + 8
RPS = 640

_sc_mesh = plsc.VectorSubcoreMesh(core_axis_name="c", subcore_axis_name="s")


@functools.partial(
    pl.kernel,
    out_type=(
        jax.ShapeDtypeStruct((NC, N, D), jnp.float32),
        jax.ShapeDtypeStruct((NW, N + 16), jnp.float32),
    ),
    mesh=_sc_mesh,
    scratch_types=[
        pltpu.VMEM((SEG, Q), jnp.int32),
        pltpu.VMEM((SEG, Q), jnp.int32),
        pltpu.VMEM((4 * Q, D), jnp.float32),
        pltpu.VMEM((N + 16,), jnp.float32),
        pltpu.VMEM_SHARED((NPR, D), jnp.float32),
        pltpu.SemaphoreType.DMA,
        pltpu.SemaphoreType.DMA,
        pltpu.SemaphoreType.DMA,
        pltpu.SemaphoreType.DMA,
    ],
    compiler_params=pltpu.CompilerParams(needs_layout_passes=False),
)
def _sc_aggregate(edges_hbm, x_hbm, agg_hbm, degp_hbm,
                  src_v, dst_v, ring, deg_v, agg_s, s0, s1, s2, s3):
    cid = lax.axis_index("c")
    sid = lax.axis_index("s")
    wid = cid * NS + sid

    zero16 = jnp.zeros((16,), jnp.float32)

    def _zring(i, carry):
        r = i // (D // 16)
        c = (i % (D // 16)) * 16
        ring[r, pl.ds(c, 16)] = zero16
        return carry

    lax.fori_loop(0, 2 * Q * (D // 16), _zring, 0)

    def _zdeg(i, carry):
        deg_v[pl.ds(i * 16, 16)] = zero16
        return carry

    lax.fori_loop(0, (N + 16) // 16, _zdeg, 0)

    base = sid * RPS
    zsrc = ring.at[pl.ds(0, 2 * Q)]

    @pl.when(sid < NS - 1)
    def _():
        for k in range(RPS // (2 * Q)):
            pltpu.sync_copy(zsrc, agg_s.at[pl.ds(base + k * 2 * Q, 2 * Q)])

    @pl.when(sid == NS - 1)
    def _():
        for k in range((N - (NS - 1) * RPS) // (2 * Q)):
            pltpu.sync_copy(
                zsrc, agg_s.at[pl.ds((NS - 1) * RPS + k * 2 * Q, 2 * Q)])
        pltpu.sync_copy(ring.at[pl.ds(0, 8)], agg_s.at[pl.ds(N, 8)])

    plsc.subcore_barrier()

    ones16 = jnp.ones((16,), jnp.float32)
    iota16 = lax.broadcasted_iota(jnp.int32, (16,), 0)
    hi8 = jnp.where(iota16 >= 8, 1, 0)

    def _deg_pair(p):
        r0 = jnp.full((16,), 2 * p, jnp.int32)
        r1 = r0 + 1
        for rv, cv in (
            (r0, iota16),
            (r0, iota16 + 16),
            (r0 + hi8, jnp.where(iota16 >= 8, iota16 - 8, iota16 + 32)),
            (r1, iota16 + 8),
            (r1, iota16 + 24),
        ):
            idx = plsc.load_gather(dst_v, [rv, cv])
            plsc.addupdate_scatter(deg_v, [idx], ones16)

    bufs = tuple(ring.at[pl.ds(q * Q, Q)] for q in range(4))
    sems = (s0, s1, s2, s3)

    def _gather_start(j, q):
        pltpu.async_copy(x_hbm.at[src_v.at[j]], bufs[q], sems[q])

    def _gather_wait(j, q):
        pltpu.make_async_copy(x_hbm.at[src_v.at[j]], bufs[q], sems[q]).wait()

    def _scat(j, q):
        pltpu.sync_copy(bufs[q], agg_s.at[dst_v.at[j]], add=True)

    def _body(i, carry):
        for t in range(4):
            j = 4 * i + t
            if t == 0:
                _deg_pair(2 * i)
            if t == 2:
                _deg_pair(2 * i + 1)
            _gather_wait(j, t)
            _scat(j, t)

            @pl.when(j + 3 < SEG)
            def _():
                _gather_start(j + 3, (t + 3) % 4)

        return carry

    for s in range(NSEG):
        pltpu.sync_copy(edges_hbm.at[0, wid, s], src_v)
        pltpu.sync_copy(edges_hbm.at[1, wid, s], dst_v)
        for j in range(3):
            _gather_start(j, j)
        lax.fori_loop(0, SEG // 4, _body, 0)

    plsc.subcore_barrier()

    @pl.when(sid < NS - 1)
    def _():
        pltpu.sync_copy(agg_s.at[pl.ds(base, RPS)],
                        agg_hbm.at[cid, pl.ds(base, RPS)])

    @pl.when(sid == NS - 1)
    def _():
        last = N - (NS - 1) * RPS
        pltpu.sync_copy(agg_s.at[pl.ds((NS - 1) * RPS, last)],
                        agg_hbm.at[cid, pl.ds((NS - 1) * RPS, last)])

    pltpu.sync_copy(deg_v, degp_hbm.at[wid])


def _deg_sum_body(degp_ref, out_ref):
    ones = jnp.ones((NW, 1), jnp.float32)
    out_ref[...] = lax.dot_general(
        degp_ref[...], ones, (((0,), (0,)), ((), ())),
        preferred_element_type=jnp.float32)


_deg_sum = pl.pallas_call(
    _deg_sum_body,
    out_shape=jax.ShapeDtypeStruct((N + 16, 1), jnp.float32),
)


BLK = 2000


def _dense_body(x_ref, agg_ref, deg_ref, wt_ref, h_ref):
    deg = jnp.clip(deg_ref[...], 1.0, None)
    aggsum = agg_ref[0] + agg_ref[1]
    neigh = aggsum / deg
    h = jnp.dot(x_ref[...], wt_ref[:D, :], preferred_element_type=jnp.float32)
    h = h + jnp.dot(neigh, wt_ref[D:, :], preferred_element_type=jnp.float32)
    h_ref[...] = jnp.maximum(h, 0.0)


_dense = pl.pallas_call(
    _dense_body,
    grid=(N // BLK,),
    in_specs=[
        pl.BlockSpec((BLK, D), lambda i: (i, 0)),
        pl.BlockSpec((NC, BLK, D), lambda i: (0, i, 0)),
        pl.BlockSpec((BLK, 1), lambda i: (i, 0)),
        pl.BlockSpec((2 * D, EMB), lambda i: (0, 0)),
    ],
    out_specs=pl.BlockSpec((BLK, EMB), lambda i: (i, 0)),
    out_shape=jax.ShapeDtypeStruct((N, EMB), jnp.float32),
)


PC = 128
PPW = 2 * B // NW
NPC = PPW // PC


@functools.partial(
    pl.kernel,
    out_type=jax.ShapeDtypeStruct((2 * B, EMB), jnp.float32),
    mesh=_sc_mesh,
    scratch_types=[
        pltpu.VMEM((NPC, PC), jnp.int32),
        pltpu.VMEM((PC, EMB), jnp.float32),
        pltpu.VMEM((PC, EMB), jnp.float32),
        pltpu.SemaphoreType.DMA,
        pltpu.SemaphoreType.DMA,
    ],
)
def _sc_pair_gather(uv_hbm, h_hbm, out_hbm, idx_v, buf0, buf1, sem0, sem1):
    cid = lax.axis_index("c")
    sid = lax.axis_index("s")
    wid = cid * NS + sid

    pltpu.sync_copy(uv_hbm.at[wid], idx_v)

    bufs = (buf0, buf1)
    sems = (sem0, sem1)
    pltpu.async_copy(h_hbm.at[idx_v.at[0]], bufs[0], sems[0])
    for j in range(NPC):
        if j + 1 < NPC:
            pltpu.async_copy(h_hbm.at[idx_v.at[j + 1]],
                             bufs[(j + 1) % 2], sems[(j + 1) % 2])
        pltpu.make_async_copy(h_hbm.at[idx_v.at[j]],
                              bufs[j % 2], sems[j % 2]).wait()
        pltpu.sync_copy(bufs[j % 2],
                        out_hbm.at[pl.ds((wid * NPC + j) * PC, PC)])


CB = 2048


def _cos_body(eu_ref, ev_ref, out_ref):
    eu = eu_ref[...]
    ev = ev_ref[...]
    num = jnp.sum(eu * ev, axis=1, keepdims=True)
    nu = jnp.clip(jnp.sqrt(jnp.sum(eu * eu, axis=1, keepdims=True)), 1e-8, None)
    nv = jnp.clip(jnp.sqrt(jnp.sum(ev * ev, axis=1, keepdims=True)), 1e-8, None)
    out_ref[...] = num / (nu * nv)


_cosine = pl.pallas_call(
    _cos_body,
    grid=(B // CB,),
    in_specs=[
        pl.BlockSpec((CB, EMB), lambda i: (i, 0)),
        pl.BlockSpec((CB, EMB), lambda i: (i + B // CB, 0)),
    ],
    out_specs=pl.BlockSpec((CB, 1), lambda i: (i, 0)),
    out_shape=jax.ShapeDtypeStruct((B, 1), jnp.float32),
)


def kernel(u, v, x, edge_index, W):
    pad = jnp.concatenate(
        [jnp.zeros((1, EPAD - E), jnp.int32),
         jnp.full((1, EPAD - E), SINK, jnp.int32)], axis=0)
    edges = jnp.concatenate([edge_index, pad], axis=1)
    edges = edges.reshape(2, NW, NSEG, SEG, Q)
    agg, degp = _sc_aggregate(edges, x)
    deg = _deg_sum(degp)[:N]
    h = _dense(x, agg, deg, W.T)
    uv = jnp.concatenate([u, v]).reshape(NW, NPC, PC)
    euv = _sc_pair_gather(uv, h)
    scores = _cosine(euv, euv).reshape(B)
    return scores


# speedup vs baseline: 2.4727x; 2.4727x over previous
import functools

import jax
import jax.numpy as jnp
from jax import lax
from jax.experimental import pallas as pl
from jax.experimental.pallas import tpu as pltpu
from jax.experimental.pallas import tpu_sc as plsc

N = 10000
E = 320000
D = 128
EMB = 128
B = 8192

NC = 2
NS = 16
NW = NC * NS

EPW = E // NW
CHUNK = 80
NCHUNK = EPW // CHUNK
SEG = 25
NSEG = NCHUNK // SEG
RPS = 640

_sc_mesh = plsc.VectorSubcoreMesh(core_axis_name="c", subcore_axis_name="s")


@functools.partial(
    pl.kernel,
    out_type=(
        jax.ShapeDtypeStruct((NC, N, D), jnp.float32),
        jax.ShapeDtypeStruct((NW, N), jnp.float32),
    ),
    mesh=_sc_mesh,
    scratch_types=[
        pltpu.VMEM((SEG, CHUNK), jnp.int32),
        pltpu.VMEM((SEG, CHUNK), jnp.int32),
        pltpu.VMEM((CHUNK, D), jnp.float32),
        pltpu.VMEM((CHUNK, D), jnp.float32),
        pltpu.VMEM((N,), jnp.float32),
        pltpu.VMEM_SHARED((N, D), jnp.float32),
        pltpu.SemaphoreType.DMA,
        pltpu.SemaphoreType.DMA,
        pltpu.SemaphoreType.DMA,
        pltpu.SemaphoreType.DMA,
    ],
    compiler_params=pltpu.CompilerParams(needs_layout_passes=False),
)
def _sc_aggregate(edges_hbm, x_hbm, agg_hbm, degp_hbm,
                  src_v, dst_v, rows0, rows1, deg_v, agg_s,
                  sem0, sem1, ssem0, ssem1):
    cid = lax.axis_index("c")
    sid = lax.axis_index("s")
    wid = cid * NS + sid

    zero16 = jnp.zeros((16,), jnp.float32)

    def _zrow(i, carry):
        r = i // (D // 16)
        c = (i % (D // 16)) * 16
        rows0[r, pl.ds(c, 16)] = zero16
        return carry

    lax.fori_loop(0, CHUNK * (D // 16), _zrow, 0)

    def _zdeg(i, carry):
        deg_v[pl.ds(i * 16, 16)] = zero16
        return carry

    lax.fori_loop(0, N // 16, _zdeg, 0)

    base = sid * RPS

    @pl.when(sid < NS - 1)
    def _():
        for k in range(RPS // CHUNK):
            pltpu.sync_copy(rows0, agg_s.at[pl.ds(base + k * CHUNK, CHUNK)])

    @pl.when(sid == NS - 1)
    def _():
        for k in range((N - (NS - 1) * RPS) // CHUNK):
            pltpu.sync_copy(
                rows0, agg_s.at[pl.ds((NS - 1) * RPS + k * CHUNK, CHUNK)])

    plsc.subcore_barrier()

    ones16 = jnp.ones((16,), jnp.float32)

    def _deg_update(j):
        for k in range(CHUNK // 16):
            idx = dst_v[j, pl.ds(k * 16, 16)]
            plsc.addupdate_scatter(deg_v, [idx], ones16)

    def _gather_start(j, rows, sem):
        pltpu.async_copy(x_hbm.at[src_v.at[j]], rows, sem)

    def _gather_wait(j, rows, sem):
        pltpu.make_async_copy(x_hbm.at[src_v.at[j]], rows, sem).wait()

    def _scat(j, rows):
        pltpu.sync_copy(rows, agg_s.at[dst_v.at[j]], add=True)

    def _body(i, carry):
        j0 = 2 * i
        _gather_start(j0 + 1, rows1, sem1)
        _deg_update(j0)
        _gather_wait(j0, rows0, sem0)
        _scat(j0, rows0)
        _gather_start(j0 + 2, rows0, sem0)
        _deg_update(j0 + 1)
        _gather_wait(j0 + 1, rows1, sem1)
        _scat(j0 + 1, rows1)
        return carry

    for s in range(NSEG):
        pltpu.sync_copy(edges_hbm.at[0, wid, s], src_v)
        pltpu.sync_copy(edges_hbm.at[1, wid, s], dst_v)
        _gather_start(0, rows0, sem0)
        lax.fori_loop(0, (SEG - 1) // 2, _body, 0)
        _deg_update(SEG - 1)
        _gather_wait(SEG - 1, rows0, sem0)
        _scat(SEG - 1, rows0)

    plsc.subcore_barrier()

    @pl.when(sid < NS - 1)
    def _():
        pltpu.sync_copy(agg_s.at[pl.ds(base, RPS)],
                        agg_hbm.at[cid, pl.ds(base, RPS)])

    @pl.when(sid == NS - 1)
    def _():
        last = N - (NS - 1) * RPS
        pltpu.sync_copy(agg_s.at[pl.ds((NS - 1) * RPS, last)],
                        agg_hbm.at[cid, pl.ds((NS - 1) * RPS, last)])

    pltpu.sync_copy(deg_v, degp_hbm.at[wid])


def _deg_sum_body(degp_ref, out_ref):
    ones = jnp.ones((NW, 1), jnp.float32)
    out_ref[...] = lax.dot_general(
        degp_ref[...], ones, (((0,), (0,)), ((), ())),
        preferred_element_type=jnp.float32)


_deg_sum = pl.pallas_call(
    _deg_sum_body,
    out_shape=jax.ShapeDtypeStruct((N, 1), jnp.float32),
)


BLK = 2000


def _dense_body(x_ref, agg_ref, deg_ref, wt_ref, h_ref):
    deg = jnp.clip(deg_ref[...], 1.0, None)
    aggsum = agg_ref[0] + agg_ref[1]
    neigh = aggsum / deg
    h = jnp.dot(x_ref[...], wt_ref[:D, :], preferred_element_type=jnp.float32)
    h = h + jnp.dot(neigh, wt_ref[D:, :], preferred_element_type=jnp.float32)
    h_ref[...] = jnp.maximum(h, 0.0)


_dense = pl.pallas_call(
    _dense_body,
    grid=(N // BLK,),
    in_specs=[
        pl.BlockSpec((BLK, D), lambda i: (i, 0)),
        pl.BlockSpec((NC, BLK, D), lambda i: (0, i, 0)),
        pl.BlockSpec((BLK, 1), lambda i: (i, 0)),
        pl.BlockSpec((2 * D, EMB), lambda i: (0, 0)),
    ],
    out_specs=pl.BlockSpec((BLK, EMB), lambda i: (i, 0)),
    out_shape=jax.ShapeDtypeStruct((N, EMB), jnp.float32),
)


PC = 128
PPW = 2 * B // NW
NPC = PPW // PC


@functools.partial(
    pl.kernel,
    out_type=jax.ShapeDtypeStruct((2 * B, EMB), jnp.float32),
    mesh=_sc_mesh,
    scratch_types=[
        pltpu.VMEM((NPC, PC), jnp.int32),
        pltpu.VMEM((PC, EMB), jnp.float32),
        pltpu.VMEM((PC, EMB), jnp.float32),
        pltpu.SemaphoreType.DMA,
        pltpu.SemaphoreType.DMA,
    ],
)
def _sc_pair_gather(uv_hbm, h_hbm, out_hbm, idx_v, buf0, buf1, sem0, sem1):
    cid = lax.axis_index("c")
    sid = lax.axis_index("s")
    wid = cid * NS + sid

    pltpu.sync_copy(uv_hbm.at[wid], idx_v)

    bufs = (buf0, buf1)
    sems = (sem0, sem1)
    pltpu.async_copy(h_hbm.at[idx_v.at[0]], bufs[0], sems[0])
    for j in range(NPC):
        if j + 1 < NPC:
            pltpu.async_copy(h_hbm.at[idx_v.at[j + 1]],
                             bufs[(j + 1) % 2], sems[(j + 1) % 2])
        pltpu.make_async_copy(h_hbm.at[idx_v.at[j]],
                              bufs[j % 2], sems[j % 2]).wait()
        pltpu.sync_copy(bufs[j % 2],
                        out_hbm.at[pl.ds((wid * NPC + j) * PC, PC)])


CB = 4096


def _cos_body(eu_ref, ev_ref, out_ref):
    eu = eu_ref[...]
    ev = ev_ref[...]
    num = jnp.sum(eu * ev, axis=1, keepdims=True)
    nu = jnp.clip(jnp.sqrt(jnp.sum(eu * eu, axis=1, keepdims=True)), 1e-8, None)
    nv = jnp.clip(jnp.sqrt(jnp.sum(ev * ev, axis=1, keepdims=True)), 1e-8, None)
    out_ref[...] = num / (nu * nv)


_cosine = pl.pallas_call(
    _cos_body,
    grid=(B // CB,),
    in_specs=[
        pl.BlockSpec((CB, EMB), lambda i: (i, 0)),
        pl.BlockSpec((CB, EMB), lambda i: (i + B // CB, 0)),
    ],
    out_specs=pl.BlockSpec((CB, 1), lambda i: (i, 0)),
    out_shape=jax.ShapeDtypeStruct((B, 1), jnp.float32),
)


def kernel(u, v, x, edge_index, W):
    edges = edge_index.reshape(2, NW, NSEG, SEG, CHUNK)
    agg, degp = _sc_aggregate(edges, x)
    deg = _deg_sum(degp)
    h = _dense(x, agg, deg, W.T)
    uv = jnp.concatenate([u, v]).reshape(NW, NPC, PC)
    euv = _sc_pair_gather(uv, h)
    scores = _cosine(euv, euv).reshape(B)
    return scores


# speedup vs baseline: 2.5333x; 1.0245x over previous
import functools

import jax
import jax.numpy as jnp
from jax import lax
from jax.experimental import pallas as pl
from jax.experimental.pallas import tpu as pltpu
from jax.experimental.pallas import tpu_sc as plsc

N = 10000
E = 320000
D = 128
EMB = 128
B = 8192

NC = 2
NS = 16
NW = NC * NS

EPW = E // NW
CHUNK = 80
NCHUNK = EPW // CHUNK
SEG = 25
NSEG = NCHUNK // SEG
RPS = 640

_sc_mesh = plsc.VectorSubcoreMesh(core_axis_name="c", subcore_axis_name="s")


@functools.partial(
    pl.kernel,
    out_type=(
        jax.ShapeDtypeStruct((NC, N, D), jnp.float32),
        jax.ShapeDtypeStruct((NW, N), jnp.float32),
    ),
    mesh=_sc_mesh,
    scratch_types=[
        pltpu.VMEM((SEG, CHUNK), jnp.int32),
        pltpu.VMEM((SEG, CHUNK), jnp.int32),
        pltpu.VMEM((CHUNK, D), jnp.float32),
        pltpu.VMEM((CHUNK, D), jnp.float32),
        pltpu.VMEM((N,), jnp.float32),
        pltpu.VMEM_SHARED((N, D), jnp.float32),
        pltpu.SemaphoreType.DMA,
        pltpu.SemaphoreType.DMA,
        pltpu.SemaphoreType.DMA,
        pltpu.SemaphoreType.DMA,
    ],
    compiler_params=pltpu.CompilerParams(needs_layout_passes=False),
)
def _sc_aggregate(edges_hbm, x_hbm, agg_hbm, degp_hbm,
                  src_v, dst_v, rows0, rows1, deg_v, agg_s,
                  sem0, sem1, ssem0, ssem1):
    cid = lax.axis_index("c")
    sid = lax.axis_index("s")
    wid = cid * NS + sid

    zero16 = jnp.zeros((16,), jnp.float32)

    def _zrow(i, carry):
        r = i // (D // 16)
        c = (i % (D // 16)) * 16
        rows0[r, pl.ds(c, 16)] = zero16
        return carry

    lax.fori_loop(0, CHUNK * (D // 16), _zrow, 0)

    def _zdeg(i, carry):
        deg_v[pl.ds(i * 16, 16)] = zero16
        return carry

    lax.fori_loop(0, N // 16, _zdeg, 0)

    base = sid * RPS

    @pl.when(sid < NS - 1)
    def _():
        for k in range(RPS // CHUNK):
            pltpu.sync_copy(rows0, agg_s.at[pl.ds(base + k * CHUNK, CHUNK)])

    @pl.when(sid == NS - 1)
    def _():
        for k in range((N - (NS - 1) * RPS) // CHUNK):
            pltpu.sync_copy(
                rows0, agg_s.at[pl.ds((NS - 1) * RPS + k * CHUNK, CHUNK)])

    plsc.subcore_barrier()

    ones16 = jnp.ones((16,), jnp.float32)

    def _deg_update(j):
        for k in range(CHUNK // 16):
            idx = dst_v[j, pl.ds(k * 16, 16)]
            plsc.addupdate_scatter(deg_v, [idx], ones16)

    def _gather_start(j, rows, sem):
        pltpu.async_copy(x_hbm.at[src_v.at[j]], rows, sem)

    def _gather_wait(j, rows, sem):
        pltpu.make_async_copy(x_hbm.at[src_v.at[j]], rows, sem).wait()

    def _scat(j, rows):
        pltpu.sync_copy(rows, agg_s.at[dst_v.at[j]], add=True)

    def _body(i, carry):
        j0 = 2 * i
        _gather_start(j0 + 1, rows1, sem1)
        _deg_update(j0)
        _gather_wait(j0, rows0, sem0)
        _scat(j0, rows0)
        _gather_start(j0 + 2, rows0, sem0)
        _deg_update(j0 + 1)
        _gather_wait(j0 + 1, rows1, sem1)
        _scat(j0 + 1, rows1)
        return carry

    for s in range(NSEG):
        pltpu.sync_copy(edges_hbm.at[0, wid, s], src_v)
        pltpu.sync_copy(edges_hbm.at[1, wid, s], dst_v)
        _gather_start(0, rows0, sem0)
        lax.fori_loop(0, (SEG - 1) // 2, _body, 0)
        _deg_update(SEG - 1)
        _gather_wait(SEG - 1, rows0, sem0)
        _scat(SEG - 1, rows0)

    plsc.subcore_barrier()

    @pl.when(sid < NS - 1)
    def _():
        pltpu.sync_copy(agg_s.at[pl.ds(base, RPS)],
                        agg_hbm.at[cid, pl.ds(base, RPS)])

    @pl.when(sid == NS - 1)
    def _():
        last = N - (NS - 1) * RPS
        pltpu.sync_copy(agg_s.at[pl.ds((NS - 1) * RPS, last)],
                        agg_hbm.at[cid, pl.ds((NS - 1) * RPS, last)])

    pltpu.sync_copy(deg_v, degp_hbm.at[wid])


def _deg_sum_body(degp_ref, out_ref):
    ones = jnp.ones((NW, 1), jnp.float32)
    out_ref[...] = lax.dot_general(
        degp_ref[...], ones, (((0,), (0,)), ((), ())),
        preferred_element_type=jnp.float32)


_deg_sum = pl.pallas_call(
    _deg_sum_body,
    out_shape=jax.ShapeDtypeStruct((N, 1), jnp.float32),
)


BLK = 2000


def _dense_body(x_ref, agg_ref, deg_ref, wt_ref, h_ref):
    deg = jnp.clip(deg_ref[...], 1.0, None)
    aggsum = agg_ref[0] + agg_ref[1]
    neigh = aggsum / deg
    h = jnp.dot(x_ref[...], wt_ref[:D, :], preferred_element_type=jnp.float32)
    h = h + jnp.dot(neigh, wt_ref[D:, :], preferred_element_type=jnp.float32)
    h_ref[...] = jnp.maximum(h, 0.0)


_dense = pl.pallas_call(
    _dense_body,
    grid=(N // BLK,),
    in_specs=[
        pl.BlockSpec((BLK, D), lambda i: (i, 0)),
        pl.BlockSpec((NC, BLK, D), lambda i: (0, i, 0)),
        pl.BlockSpec((BLK, 1), lambda i: (i, 0)),
        pl.BlockSpec((2 * D, EMB), lambda i: (0, 0)),
    ],
    out_specs=pl.BlockSpec((BLK, EMB), lambda i: (i, 0)),
    out_shape=jax.ShapeDtypeStruct((N, EMB), jnp.float32),
)


PC = 128
PPW = 2 * B // NW
NPC = PPW // PC


@functools.partial(
    pl.kernel,
    out_type=jax.ShapeDtypeStruct((2 * B, EMB), jnp.float32),
    mesh=_sc_mesh,
    scratch_types=[
        pltpu.VMEM((NPC, PC), jnp.int32),
        pltpu.VMEM((PC, EMB), jnp.float32),
        pltpu.VMEM((PC, EMB), jnp.float32),
        pltpu.SemaphoreType.DMA,
        pltpu.SemaphoreType.DMA,
    ],
)
def _sc_pair_gather(uv_hbm, h_hbm, out_hbm, idx_v, buf0, buf1, sem0, sem1):
    cid = lax.axis_index("c")
    sid = lax.axis_index("s")
    wid = cid * NS + sid

    pltpu.sync_copy(uv_hbm.at[wid], idx_v)

    bufs = (buf0, buf1)
    sems = (sem0, sem1)
    pltpu.async_copy(h_hbm.at[idx_v.at[0]], bufs[0], sems[0])
    for j in range(NPC):
        if j + 1 < NPC:
            pltpu.async_copy(h_hbm.at[idx_v.at[j + 1]],
                             bufs[(j + 1) % 2], sems[(j + 1) % 2])
        pltpu.make_async_copy(h_hbm.at[idx_v.at[j]],
                              bufs[j % 2], sems[j % 2]).wait()
        pltpu.sync_copy(bufs[j % 2],
                        out_hbm.at[pl.ds((wid * NPC + j) * PC, PC)])


CB = 4096


def _cos_body(eu_ref, ev_ref, out_ref):
    eu = eu_ref[...]
    ev = ev_ref[...]
    num = jnp.sum(eu * ev, axis=1, keepdims=True)
    nu = jnp.clip(jnp.sqrt(jnp.sum(eu * eu, axis=1, keepdims=True)), 1e-8, None)
    nv = jnp.clip(jnp.sqrt(jnp.sum(ev * ev, axis=1, keepdims=True)), 1e-8, None)
    out_ref[...] = jnp.transpose(num / (nu * nv))


_cosine = pl.pallas_call(
    _cos_body,
    grid=(B // CB,),
    in_specs=[
        pl.BlockSpec((CB, EMB), lambda i: (i, 0)),
        pl.BlockSpec((CB, EMB), lambda i: (i + B // CB, 0)),
    ],
    out_specs=pl.BlockSpec((1, CB), lambda i: (0, i)),
    out_shape=jax.ShapeDtypeStruct((1, B), jnp.float32),
)


def kernel(u, v, x, edge_index, W):
    edges = edge_index.reshape(2, NW, NSEG, SEG, CHUNK)
    agg, degp = _sc_aggregate(edges, x)
    deg = _deg_sum(degp)
    h = _dense(x, agg, deg, W.T)
    uv = jnp.concatenate([u, v]).reshape(NW, NPC, PC)
    euv = _sc_pair_gather(uv, h)
    scores = _cosine(euv, euv).reshape(B)
    return scores
